# Initial kernel scaffold; baseline (speedup 1.0000x reference)
import jax
import jax.numpy as jnp
from jax.experimental import pallas as pl


def kernel(src, static, times, lengths, R_u, emb_W, emb_b, g1_W, g1_asrc, g1_adst, g2_W, g2_asrc, g2_adst, t_Wq, t_bq, t_Wk, t_bk, t_Wv, t_bv, t_Wo, t_bo, t_W1, t_b1, t_W2, t_b2, ln1_g, ln1_b, ln2_g, ln2_b, mlp_W1, mlp_b1, mlp_W2, mlp_b2):
    raise NotImplementedError("write your pallas kernel here")


# speedup vs baseline: 3.9550x; 3.9550x over previous
import functools
import math

import jax
import jax.numpy as jnp
from jax.experimental import pallas as pl
from jax.experimental.pallas import tpu as pltpu

D_INP = 36
D_OB = 4
T = 215
B = 128
D_PE = 16
MAXV = 100.0
NHEAD = 4
D = D_INP * D_OB + D_PE
NHID = 128
NLAYERS = 2
DSTATIC = 9
NCLS = 2
DFINAL = D + D_INP
G = T * D_OB

BLK = 8
N = D_INP
BN = BLK * N


def _gat_kernel(x_ref, scale_ref, w1_ref, as1_ref, ad1_ref,
                w2_ref, as2_ref, ad2_ref, bm_ref, out_ref):
    bm = bm_ref[...]
    x = jnp.maximum(x_ref[...] * scale_ref[...], 0.0)

    def attn(h, a_s_row, a_d_row, ew):
        ss = jnp.sum(h * a_s_row, axis=1, keepdims=True)
        dd = jax.lax.dot_general(a_d_row, h, (((1,), (1,)), ((), ())),
                                 preferred_element_type=jnp.float32)
        s = ss + dd
        s = jnp.where(s >= 0, s, 0.2 * s)
        if ew is not None:
            s = s * ew
        sm = jnp.where(bm > 0, s, -1e30)
        mx = jnp.max(sm, axis=0, keepdims=True)
        e = jnp.exp(sm - mx) * bm
        den = jnp.sum(e, axis=0, keepdims=True)
        alpha = e / (den + 1e-16)
        out = jax.lax.dot_general(alpha, h, (((0,), (0,)), ((), ())),
                                  preferred_element_type=jnp.float32)
        return out, alpha

    h1 = jnp.dot(x, w1_ref[...], preferred_element_type=jnp.float32)
    o1, a1 = attn(h1, as1_ref[...], ad1_ref[...], None)
    h2 = jnp.dot(o1, w2_ref[...], preferred_element_type=jnp.float32)
    o2, _ = attn(h2, as2_ref[...], ad2_ref[...], a1)
    out_ref[...] = o2


def _tfm_kernel(len_ref, r_ref, wq_ref, bq_ref, wk_ref, bk_ref, wv_ref,
                bv_ref, wo_ref, bo_ref, w1_ref, b1_ref, w2_ref, b2_ref,
                l1g_ref, l1b_ref, l2g_ref, l2b_ref, out_ref):
    b = pl.program_id(0)
    L = len_ref[b]
    r = r_ref[0]
    colmask = jax.lax.broadcasted_iota(jnp.int32, (1, T), 1) >= L
    dh = D // NHEAD
    inv_sqrt_dh = 1.0 / math.sqrt(dh)

    def layer_norm(x, g, bb):
        m = jnp.mean(x, axis=-1, keepdims=True)
        v = jnp.mean((x - m) ** 2, axis=-1, keepdims=True)
        return (x - m) / jnp.sqrt(v + 1e-5) * g + bb

    for l in range(NLAYERS):
        q = jnp.dot(r, wq_ref[l], preferred_element_type=jnp.float32) + bq_ref[l:l + 1]
        k = jnp.dot(r, wk_ref[l], preferred_element_type=jnp.float32) + bk_ref[l:l + 1]
        v = jnp.dot(r, wv_ref[l], preferred_element_type=jnp.float32) + bv_ref[l:l + 1]
        heads = []
        for h in range(NHEAD):
            qh = jax.lax.slice(q, (0, h * dh), (T, (h + 1) * dh))
            kh = jax.lax.slice(k, (0, h * dh), (T, (h + 1) * dh))
            vh = jax.lax.slice(v, (0, h * dh), (T, (h + 1) * dh))
            sc = jax.lax.dot_general(qh, kh, (((1,), (1,)), ((), ())),
                                     preferred_element_type=jnp.float32)
            sc = sc * inv_sqrt_dh
            sc = jnp.where(colmask, -1e9, sc)
            mx = jnp.max(sc, axis=-1, keepdims=True)
            e = jnp.exp(sc - mx)
            at = e / jnp.sum(e, axis=-1, keepdims=True)
            heads.append(jnp.dot(at, vh, preferred_element_type=jnp.float32))
        o = jnp.concatenate(heads, axis=1)
        o = jnp.dot(o, wo_ref[l], preferred_element_type=jnp.float32) + bo_ref[l:l + 1]
        r = layer_norm(r + o, l1g_ref[l:l + 1], l1b_ref[l:l + 1])
        ff = jnp.maximum(
            jnp.dot(r, w1_ref[l], preferred_element_type=jnp.float32) + b1_ref[l:l + 1],
            0.0)
        ff = jnp.dot(ff, w2_ref[l], preferred_element_type=jnp.float32) + b2_ref[l:l + 1]
        r = layer_norm(r + ff, l2g_ref[l:l + 1], l2b_ref[l:l + 1])

    tmask = (jax.lax.broadcasted_iota(jnp.int32, (T, 1), 0) < L).astype(jnp.float32)
    pooled = jnp.sum(r * tmask, axis=0, keepdims=True)
    out_ref[0] = pooled / (L.astype(jnp.float32) + 1.0)


def _head_kernel(pooled_ref, static_ref, embw_ref, embb_ref,
                 w1_ref, b1_ref, w2_ref, b2_ref, out_ref):
    emb = jnp.dot(static_ref[...], embw_ref[...],
                  preferred_element_type=jnp.float32) + embb_ref[...]
    feat = jnp.concatenate([pooled_ref[...], emb], axis=1)
    hmid = jnp.maximum(
        jnp.dot(feat, w1_ref[...], preferred_element_type=jnp.float32)
        + b1_ref[...], 0.0)
    out_ref[...] = jnp.dot(hmid, w2_ref[...],
                           preferred_element_type=jnp.float32) + b2_ref[...]


def _full(spec_shape=None):
    return pl.BlockSpec(spec_shape, lambda *_: tuple(0 for _ in spec_shape)) \
        if spec_shape else None


@jax.jit
def kernel(src, static, times, lengths, R_u, emb_W, emb_b, g1_W, g1_asrc,
           g1_adst, g2_W, g2_asrc, g2_adst, t_Wq, t_bq, t_Wk, t_bk, t_Wv,
           t_bv, t_Wo, t_bo, t_W1, t_b1, t_W2, t_b2, ln1_g, ln1_b, ln2_g,
           ln2_b, mlp_W1, mlp_b1, mlp_W2, mlp_b2):
    f32 = jnp.float32

    x0 = src[:, :, :D_INP].transpose(1, 2, 0)
    xrep = jnp.broadcast_to(x0[..., None], (B, D_INP, T, D_OB))
    x2d = xrep.reshape(B * D_INP, G)
    scale = jnp.tile(R_u.reshape(D_INP, D_OB), (BLK, T))

    ids = jnp.arange(BN, dtype=jnp.int32) // N
    bm = (ids[:, None] == ids[None, :]).astype(f32)

    cparams = pltpu.CompilerParams(dimension_semantics=("arbitrary",))

    gat_out = pl.pallas_call(
        _gat_kernel,
        grid=(B // BLK,),
        in_specs=[
            pl.BlockSpec((BN, G), lambda i: (i, 0)),
            _full((BN, G)),
            _full((G, G)),
            _full((1, G)),
            _full((1, G)),
            _full((G, G)),
            _full((1, G)),
            _full((1, G)),
            _full((BN, BN)),
        ],
        out_specs=pl.BlockSpec((BN, G), lambda i: (i, 0)),
        out_shape=jax.ShapeDtypeStruct((B * D_INP, G), f32),
        compiler_params=cparams,
    )(x2d, scale, g1_W, g1_asrc.reshape(1, G), g1_adst.reshape(1, G),
      g2_W, g2_asrc.reshape(1, G), g2_adst.reshape(1, G), bm)

    gat_t = gat_out.reshape(B, D_INP, T, D_OB).transpose(0, 2, 1, 3)
    gat_t = gat_t.reshape(B, T, D_INP * D_OB)
    timescales = (float(T) ** jnp.linspace(0.0, 1.0, D_PE // 2)) * MAXV
    scaled = times.transpose(1, 0)[:, :, None] / timescales[None, None, :]
    pe = jnp.concatenate([jnp.sin(scaled), jnp.cos(scaled)], axis=-1)
    r0 = jnp.concatenate([gat_t, pe], axis=2)

    pooled = pl.pallas_call(
        _tfm_kernel,
        grid=(B,),
        in_specs=[
            pl.BlockSpec(memory_space=pltpu.SMEM),
            pl.BlockSpec((1, T, D), lambda i: (i, 0, 0)),
            _full((NLAYERS, D, D)), _full((NLAYERS, D)),
            _full((NLAYERS, D, D)), _full((NLAYERS, D)),
            _full((NLAYERS, D, D)), _full((NLAYERS, D)),
            _full((NLAYERS, D, D)), _full((NLAYERS, D)),
            _full((NLAYERS, D, NHID)), _full((NLAYERS, NHID)),
            _full((NLAYERS, NHID, D)), _full((NLAYERS, D)),
            _full((NLAYERS, D)), _full((NLAYERS, D)),
            _full((NLAYERS, D)), _full((NLAYERS, D)),
        ],
        out_specs=pl.BlockSpec((1, 1, D), lambda i: (i, 0, 0)),
        out_shape=jax.ShapeDtypeStruct((B, 1, D), f32),
        compiler_params=cparams,
    )(lengths, r0, t_Wq, t_bq, t_Wk, t_bk, t_Wv, t_bv, t_Wo, t_bo,
      t_W1, t_b1, t_W2, t_b2, ln1_g, ln1_b, ln2_g, ln2_b)
    pooled = pooled.reshape(B, D)

    logits = pl.pallas_call(
        _head_kernel,
        grid=(1,),
        in_specs=[
            _full((B, D)), _full((B, DSTATIC)),
            _full((DSTATIC, D_INP)), _full((1, D_INP)),
            _full((DFINAL, DFINAL)), _full((1, DFINAL)),
            _full((DFINAL, NCLS)), _full((1, NCLS)),
        ],
        out_specs=_full((B, NCLS)),
        out_shape=jax.ShapeDtypeStruct((B, NCLS), f32),
        compiler_params=cparams,
    )(pooled, static, emb_W, emb_b.reshape(1, D_INP),
      mlp_W1, mlp_b1.reshape(1, DFINAL), mlp_W2, mlp_b2.reshape(1, NCLS))

    return logits
